# baseline (speedup 1.0000x reference)
import functools

import jax
import jax.numpy as jnp
from jax import lax
from jax.experimental import pallas as pl
from jax.experimental.pallas import tpu as pltpu
from jax.experimental.pallas import tpu_sc as plsc

_V = 1000000
_B = 4096
_H = 200
_D = 64
_O = 5

_CHUNK_BOUNDS = (0, 104, 200)

_NBUF = 4

_TW = 16
_TB = 1000
_NG = 8
_COLS_PER_G = _V // _NG


def _sc_transpose(table_t):
  info = plsc.get_sparse_core_info()
  nc, ns = info.num_cores, info.num_subcores

  mesh = plsc.VectorSubcoreMesh(core_axis_name="c", subcore_axis_name="s")
  nblocks = _COLS_PER_G // _TB

  @functools.partial(
      pl.kernel,
      out_type=jax.ShapeDtypeStruct((_V, _D), jnp.float32),
      mesh=mesh,
      scratch_types=[
          pltpu.VMEM((2, _TW, _TB), jnp.float32),
          pltpu.VMEM((2, _TB, _TW), jnp.float32),
          pltpu.SemaphoreType.DMA,
          pltpu.SemaphoreType.DMA,
          pltpu.SemaphoreType.DMA,
          pltpu.SemaphoreType.DMA,
      ],
      compiler_params=pltpu.CompilerParams(
          use_tc_tiling_on_sc=False, needs_layout_passes=False),
  )
  def k(tt_hbm, out_hbm, in_v, out_v, si0, si1, so0, so1):
    wid = lax.axis_index("s") * nc + lax.axis_index("c")
    g = wid % _NG
    d0 = (wid // _NG) * _TW
    c_base = g * _COLS_PER_G
    sis = (si0, si1)
    sos = (so0, so1)

    def in_desc(i, nb):
      return pltpu.make_async_copy(
          tt_hbm.at[pl.ds(d0, _TW), pl.ds(c_base + i * _TB, _TB)],
          in_v.at[nb], sis[nb])

    def out_desc(i, nb):
      return pltpu.make_async_copy(
          out_v.at[nb],
          out_hbm.at[pl.ds(c_base + i * _TB, _TB), pl.ds(d0, _TW)],
          sos[nb])

    rowidx = lax.iota(jnp.int32, 16)

    def transpose_block(nb):
      @pl.loop(0, _TB, unroll=8)
      def _(col):
        colvec = jnp.full((16,), col, jnp.int32)
        vals = plsc.load_gather(in_v.at[nb], [rowidx, colvec])
        out_v[nb, col, pl.ds(0, 16)] = vals

    main = (nblocks // 2) * 2

    in_desc(0, 0).start()

    @pl.loop(0, main, step=2)
    def _(i):
      for nb in range(2):
        blk = i + nb

        @pl.when(blk + 1 < nblocks)
        def _():
          in_desc(blk + 1, 1 - nb).start()

        in_desc(blk, nb).wait()

        @pl.when(blk >= 2)
        def _():
          out_desc(blk - 2, nb).wait()

        transpose_block(nb)
        out_desc(blk, nb).start()

    for blk in range(main, nblocks):
      nb = blk % 2
      in_desc(blk, nb).wait()
      out_desc(blk - 2, nb).wait()
      transpose_block(nb)
      out_desc(blk, nb).start()

    out_desc(nblocks - 2, (nblocks - 2) % 2).wait()
    out_desc(nblocks - 1, (nblocks - 1) % 2).wait()

  return k(table_t)


def _sc_pool(idx2, table_rm):
  info = plsc.get_sparse_core_info()
  nc, ns = info.num_cores, info.num_subcores
  nw = nc * ns
  b_per_w = _B // nw

  mesh = plsc.VectorSubcoreMesh(core_axis_name="c", subcore_axis_name="s")

  @functools.partial(
      pl.kernel,
      out_type=jax.ShapeDtypeStruct((_B, _D), jnp.float32),
      mesh=mesh,
      scratch_types=[
          pltpu.VMEM((b_per_w, _H), jnp.int32),
          pltpu.VMEM((_NBUF, _H, _D), jnp.float32),
          pltpu.VMEM((b_per_w, _D), jnp.float32),
      ] + [pltpu.SemaphoreType.DMA] * _NBUF,
      compiler_params=pltpu.CompilerParams(use_tc_tiling_on_sc=False),
  )
  def k(idx_hbm, table_hbm, out_hbm, idx_v, rows_v, pooled_v, *sems):
    wid = lax.axis_index("s") * nc + lax.axis_index("c")
    base = wid * b_per_w
    pltpu.sync_copy(idx_hbm.at[pl.ds(base, b_per_w)], idx_v)

    zero = jnp.zeros((16,), jnp.float32)

    def gather_descs(row, nb):
      return [
          pltpu.make_async_copy(
              table_hbm.at[idx_v.at[row, pl.ds(lo, hi - lo)]],
              rows_v.at[nb, pl.ds(lo, hi - lo)],
              sems[nb],
          )
          for lo, hi in zip(_CHUNK_BOUNDS[:-1], _CHUNK_BOUNDS[1:])
      ]

    def pool_row(row, nb):
      @pl.loop(0, _H, init_carry=(zero, zero, zero, zero), unroll=8)
      def acc_loop(r, carry):
        a0, a1, a2, a3 = carry
        a0 = a0 + rows_v[nb, r, pl.ds(0, 16)]
        a1 = a1 + rows_v[nb, r, pl.ds(16, 16)]
        a2 = a2 + rows_v[nb, r, pl.ds(32, 16)]
        a3 = a3 + rows_v[nb, r, pl.ds(48, 16)]
        return a0, a1, a2, a3

      a0, a1, a2, a3 = acc_loop
      pooled_v[row, pl.ds(0, 16)] = a0
      pooled_v[row, pl.ds(16, 16)] = a1
      pooled_v[row, pl.ds(32, 16)] = a2
      pooled_v[row, pl.ds(48, 16)] = a3

    for nb in range(_NBUF):
      for d in gather_descs(nb, nb):
        d.start()

    @pl.loop(0, b_per_w, step=_NBUF)
    def _(i):
      for nb in range(_NBUF):
        row = i + nb
        for d in gather_descs(row, nb):
          d.wait()
        pool_row(row, nb)

        @pl.when(row + _NBUF < b_per_w)
        def _():
          for d in gather_descs(row + _NBUF, nb):
            d.start()

    pltpu.sync_copy(pooled_v, out_hbm.at[pl.ds(base, b_per_w)])

  return k(idx2, table_rm)


def _tc_linear(pooled, bias2, W, b2):
  def body(pooled_ref, bias_ref, w_ref, b_ref, out_ref):
    x = pooled_ref[...] + bias_ref[...]
    out_ref[...] = (
        jnp.dot(x, w_ref[...], preferred_element_type=jnp.float32)
        + b_ref[...]
    )

  return pl.pallas_call(
      body,
      out_shape=jax.ShapeDtypeStruct((_B, _O), jnp.float32),
  )(pooled, bias2, W, b2)


def kernel(inputs, embed_table, bias, W, b):
  table_rm = _sc_transpose(embed_table.T)
  pooled = _sc_pool(inputs.astype(jnp.int32), table_rm)
  return _tc_linear(pooled, bias.reshape(1, _D), W, b.reshape(1, _O))


# speedup vs baseline: 8.2975x; 8.2975x over previous
import functools

import jax
import jax.numpy as jnp
from jax import lax
from jax.experimental import pallas as pl
from jax.experimental.pallas import tpu as pltpu
from jax.experimental.pallas import tpu_sc as plsc

_B = 4096
_H = 200
_D = 64
_O = 5

_CHUNK_BOUNDS = (0, 104, 200)
_CHUNKS_PER_ROW = len(_CHUNK_BOUNDS) - 1

_NBUF = 4


def _sc_pool(idx2, embed_table):
  info = plsc.get_sparse_core_info()
  nc, ns = info.num_cores, info.num_subcores
  nw = nc * ns
  b_per_w = _B // nw

  mesh = plsc.VectorSubcoreMesh(core_axis_name="c", subcore_axis_name="s")

  @functools.partial(
      pl.kernel,
      out_type=jax.ShapeDtypeStruct((_B, _D), jnp.float32),
      mesh=mesh,
      scratch_types=[
          pltpu.VMEM((b_per_w, _H), jnp.int32),
          pltpu.VMEM((_NBUF, _H, _D), jnp.float32),
          pltpu.VMEM((b_per_w, _D), jnp.float32),
      ] + [pltpu.SemaphoreType.DMA] * _NBUF,
      compiler_params=pltpu.CompilerParams(use_tc_tiling_on_sc=False),
  )
  def k(idx_hbm, table_hbm, out_hbm, idx_v, rows_v, pooled_v, *sems):
    wid = lax.axis_index("s") * nc + lax.axis_index("c")
    base = wid * b_per_w
    pltpu.sync_copy(idx_hbm.at[pl.ds(base, b_per_w)], idx_v)

    zero = jnp.zeros((16,), jnp.float32)

    def gather_descs(row, nb):
      return [
          pltpu.make_async_copy(
              table_hbm.at[idx_v.at[row, pl.ds(lo, hi - lo)]],
              rows_v.at[nb, pl.ds(lo, hi - lo)],
              sems[nb],
          )
          for lo, hi in zip(_CHUNK_BOUNDS[:-1], _CHUNK_BOUNDS[1:])
      ]

    def pool_row(row, nb):
      @pl.loop(0, _H, init_carry=(zero, zero, zero, zero), unroll=8)
      def acc_loop(r, carry):
        a0, a1, a2, a3 = carry
        a0 = a0 + rows_v[nb, r, pl.ds(0, 16)]
        a1 = a1 + rows_v[nb, r, pl.ds(16, 16)]
        a2 = a2 + rows_v[nb, r, pl.ds(32, 16)]
        a3 = a3 + rows_v[nb, r, pl.ds(48, 16)]
        return a0, a1, a2, a3

      a0, a1, a2, a3 = acc_loop
      pooled_v[row, pl.ds(0, 16)] = a0
      pooled_v[row, pl.ds(16, 16)] = a1
      pooled_v[row, pl.ds(32, 16)] = a2
      pooled_v[row, pl.ds(48, 16)] = a3

    for nb in range(_NBUF):
      for d in gather_descs(nb, nb):
        d.start()

    @pl.loop(0, b_per_w, step=_NBUF)
    def _(i):
      for nb in range(_NBUF):
        row = i + nb
        for d in gather_descs(row, nb):
          d.wait()
        pool_row(row, nb)

        @pl.when(row + _NBUF < b_per_w)
        def _():
          for d in gather_descs(row + _NBUF, nb):
            d.start()

    pltpu.sync_copy(pooled_v, out_hbm.at[pl.ds(base, b_per_w)])

  return k(idx2, embed_table)


def _tc_linear(pooled, bias2, W, b2):
  def body(pooled_ref, bias_ref, w_ref, b_ref, out_ref):
    x = pooled_ref[...] + bias_ref[...]
    out_ref[...] = (
        jnp.dot(x, w_ref[...], preferred_element_type=jnp.float32)
        + b_ref[...]
    )

  return pl.pallas_call(
      body,
      out_shape=jax.ShapeDtypeStruct((_B, _O), jnp.float32),
  )(pooled, bias2, W, b2)


def kernel(inputs, embed_table, bias, W, b):
  pooled = _sc_pool(inputs.astype(jnp.int32), embed_table)
  return _tc_linear(pooled, bias.reshape(1, _D), W, b.reshape(1, _O))
